# Initial kernel scaffold; baseline (speedup 1.0000x reference)
import jax
import jax.numpy as jnp
from jax.experimental import pallas as pl


def kernel(x, edge_index, batch, W1, b1, W2, b2):
    raise NotImplementedError("write your pallas kernel here")


# speedup vs baseline: 5.3302x; 5.3302x over previous
import functools

import jax
import jax.numpy as jnp
from jax import lax
from jax.experimental import pallas as pl
from jax.experimental.pallas import tpu as pltpu
from jax.experimental.pallas import tpu_sc as plsc

EPS = 1e-7
MIN_NORM = 1e-15
MAX_NORM = 1e6

N = 10000
E = 320000
D1 = 128
DEG_W = 16
DW1 = D1 + DEG_W
D2 = 64
NG = 128

NCORES = 2
NSUB = 16
EDGES_PER_SC = E // NCORES
EDGES_PER_TILE = E // (NCORES * NSUB)
K = 80
ROWS_PER_TILE = N // NSUB

BLK = 1000


def _sinh(t):
    e = jnp.exp(t)
    return 0.5 * (e - 1.0 / e)


def _cosh(t):
    e = jnp.exp(t)
    return 0.5 * (e + 1.0 / e)


def _arcosh(t):
    return jnp.log(t + jnp.sqrt(jnp.clip(t * t - 1.0, 1e-15, None)))


def _rownorm(v):
    return jnp.sqrt(jnp.sum(v * v, axis=1, keepdims=True))


def _hyp_linear_tangent(mu, bb):
    xn = jnp.maximum(_rownorm(mu), MIN_NORM)
    y = _sinh(xn) * mu / xn
    x0 = jnp.sqrt(jnp.clip(1.0 + jnp.sum(y * y, axis=1, keepdims=True), EPS, None))
    yn = jnp.maximum(_rownorm(y), MIN_NORM)
    yhat = y / yn
    alpha = jnp.sum(yhat * bb, axis=1, keepdims=True)
    w = bb - alpha * (1.0 - x0) * yhat
    first = jnp.sum(y * w, axis=1, keepdims=True) / jnp.clip(x0, EPS, None)
    md = jnp.sum(w * w, axis=1, keepdims=True) - first * first
    normu = jnp.minimum(jnp.sqrt(jnp.clip(md, EPS, None)), MAX_NORM)
    theta = jnp.maximum(normu, MIN_NORM)
    h = _cosh(theta) * y + _sinh(theta) * w / theta
    hn2 = jnp.sum(h * h, axis=1, keepdims=True)
    h0 = jnp.sqrt(jnp.clip(1.0 + hn2, EPS, None))
    y2n = jnp.maximum(jnp.sqrt(hn2), MIN_NORM)
    return _arcosh(jnp.clip(h0, 1.0 + EPS, None)) * h / y2n


def _layer1_body(x_ref, w1t_ref, b1_ref, out_ref):
    x = x_ref[...]
    sp = (lax.broadcasted_iota(jnp.int32, (1, D1), 1) != 0).astype(jnp.float32)
    x0 = jnp.sum(x * (1.0 - sp), axis=1, keepdims=True)
    y = x * sp
    yn = jnp.maximum(_rownorm(y), MIN_NORM)
    t0 = _arcosh(jnp.clip(x0, 1.0 + EPS, None)) * y / yn
    mu = jnp.dot(t0, w1t_ref[...], preferred_element_type=jnp.float32) * sp
    bb = b1_ref[...] * sp
    t = _hyp_linear_tangent(mu, bb)
    ones = jnp.ones((t.shape[0], DEG_W), jnp.float32)
    out_ref[...] = jnp.concatenate([t, ones], axis=1)


def _layer2_body(p_ref, w2t_ref, b2_ref, out_ref):
    p = p_ref[...]
    s = p[0] + p[1]
    lane = lax.broadcasted_iota(jnp.int32, (1, DW1), 1)
    deg = jnp.sum(s * (lane == D1).astype(jnp.float32), axis=1, keepdims=True)
    a = s[:, :D1] / jnp.maximum(deg, 1.0)
    r = jnp.maximum(a, 0.0)
    sp = (lax.broadcasted_iota(jnp.int32, (1, D2), 1) != 0).astype(jnp.float32)
    mu = jnp.dot(r, w2t_ref[...], preferred_element_type=jnp.float32) * sp
    bb = b2_ref[...] * sp
    out_ref[...] = _hyp_linear_tangent(mu, bb)


def _pool_body(q_ref, dp_ref, batch_ref, out_ref):
    q = q_ref[...]
    s = q[0] + q[1]
    dp = dp_ref[...]
    dsum = dp[0] + dp[1]
    lane16 = lax.broadcasted_iota(jnp.int32, (1, DEG_W), 1)
    deg = jnp.sum(dsum * (lane16 == 0).astype(jnp.float32), axis=1, keepdims=True)
    a = s / jnp.maximum(deg, 1.0)
    b = batch_ref[...]
    gid = lax.broadcasted_iota(jnp.int32, (NG, N), 0)
    oh = (gid == b).astype(jnp.float32)
    gsum = jnp.dot(oh, a, preferred_element_type=jnp.float32)
    cnt = jnp.sum(oh, axis=1, keepdims=True)
    g = gsum / jnp.maximum(cnt, 1.0)
    sh = g - jnp.max(g, axis=1, keepdims=True)
    out_ref[...] = sh - jnp.log(jnp.sum(jnp.exp(sh), axis=1, keepdims=True))


def _make_sc_agg(width):
    mesh = plsc.VectorSubcoreMesh(core_axis_name="c", subcore_axis_name="s")

    @functools.partial(
        pl.kernel,
        out_type=jax.ShapeDtypeStruct((NCORES * N, width), jnp.float32),
        mesh=mesh,
        scratch_types=[
            pltpu.VMEM((K,), jnp.int32),
            pltpu.VMEM((K,), jnp.int32),
            pltpu.VMEM((K, width), jnp.float32),
            pltpu.VMEM_SHARED((N, width), jnp.float32),
            pltpu.SemaphoreType.DMA,
        ],
        compiler_params=pltpu.CompilerParams(use_tc_tiling_on_sc=False),
    )
    def agg(table, src, dst, zeros, out, src_v, dst_v, rows_v, acc, sem):
        c = lax.axis_index("c")
        s = lax.axis_index("s")
        rb = s * ROWS_PER_TILE
        pltpu.sync_copy(zeros.at[pl.ds(rb, ROWS_PER_TILE)],
                        acc.at[pl.ds(rb, ROWS_PER_TILE)])
        plsc.subcore_barrier()
        ebase = c * EDGES_PER_SC + s * EDGES_PER_TILE

        def body(i, carry):
            eb = ebase + i * K
            pltpu.sync_copy(src.at[pl.ds(eb, K)], src_v)
            pltpu.sync_copy(dst.at[pl.ds(eb, K)], dst_v)
            pltpu.async_copy(table.at[src_v], rows_v, sem).wait()
            pltpu.sync_copy(rows_v, acc.at[dst_v], add=True)
            return carry

        lax.fori_loop(0, EDGES_PER_TILE // K, body, 0)
        plsc.subcore_barrier()
        pltpu.sync_copy(acc.at[pl.ds(rb, ROWS_PER_TILE)],
                        out.at[pl.ds(c * N + rb, ROWS_PER_TILE)])

    return agg


_make_sc_agg = functools.lru_cache(maxsize=None)(_make_sc_agg)


def kernel(x, edge_index, batch, W1, b1, W2, b2):
    src = edge_index[0]
    dst = edge_index[1]

    t1 = pl.pallas_call(
        _layer1_body,
        grid=(N // BLK,),
        in_specs=[
            pl.BlockSpec((BLK, D1), lambda i: (i, 0)),
            pl.BlockSpec((D1, D1), lambda i: (0, 0)),
            pl.BlockSpec((1, D1), lambda i: (0, 0)),
        ],
        out_specs=pl.BlockSpec((BLK, DW1), lambda i: (i, 0)),
        out_shape=jax.ShapeDtypeStruct((N, DW1), jnp.float32),
    )(x, W1.T, b1.reshape(1, -1))

    p1 = _make_sc_agg(DW1)(t1, src, dst, jnp.zeros((N, DW1), jnp.float32))
    p1 = p1.reshape(NCORES, N, DW1)

    t2 = pl.pallas_call(
        _layer2_body,
        grid=(N // BLK,),
        in_specs=[
            pl.BlockSpec((NCORES, BLK, DW1), lambda i: (0, i, 0)),
            pl.BlockSpec((D1, D2), lambda i: (0, 0)),
            pl.BlockSpec((1, D2), lambda i: (0, 0)),
        ],
        out_specs=pl.BlockSpec((BLK, D2), lambda i: (i, 0)),
        out_shape=jax.ShapeDtypeStruct((N, D2), jnp.float32),
    )(p1, W2.T, b2.reshape(1, -1))

    p2 = _make_sc_agg(D2)(t2, src, dst, jnp.zeros((N, D2), jnp.float32))
    p2 = p2.reshape(NCORES, N, D2)

    out = pl.pallas_call(
        _pool_body,
        in_specs=[
            pl.BlockSpec((NCORES, N, D2), lambda: (0, 0, 0)),
            pl.BlockSpec((NCORES, N, DEG_W), lambda: (0, 0, 0)),
            pl.BlockSpec((1, N), lambda: (0, 0)),
        ],
        out_specs=pl.BlockSpec((NG, D2), lambda: (0, 0)),
        out_shape=jax.ShapeDtypeStruct((NG, D2), jnp.float32),
    )(p2, p1[:, :, D1:], batch.reshape(1, -1))

    return out


# speedup vs baseline: 12.2703x; 2.3020x over previous
import functools

import jax
import jax.numpy as jnp
from jax import lax
from jax.experimental import pallas as pl
from jax.experimental.pallas import tpu as pltpu
from jax.experimental.pallas import tpu_sc as plsc

EPS = 1e-7
MIN_NORM = 1e-15
MAX_NORM = 1e6

N = 10000
E = 320000
D1 = 128
DEG_W = 16
DW1 = D1 + DEG_W
D2 = 64
NG = 128

NCORES = 2
NSUB = 16
EDGES_PER_SC = E // NCORES
EDGES_PER_TILE = E // (NCORES * NSUB)
K = 80
ROWS_PER_TILE = N // NSUB

BLK = 1000


def _sinh(t):
    e = jnp.exp(t)
    return 0.5 * (e - 1.0 / e)


def _cosh(t):
    e = jnp.exp(t)
    return 0.5 * (e + 1.0 / e)


def _arcosh(t):
    return jnp.log(t + jnp.sqrt(jnp.clip(t * t - 1.0, 1e-15, None)))


def _rownorm(v):
    return jnp.sqrt(jnp.sum(v * v, axis=1, keepdims=True))


def _hyp_linear_tangent(mu, bb):
    xn = jnp.maximum(_rownorm(mu), MIN_NORM)
    y = _sinh(xn) * mu / xn
    x0 = jnp.sqrt(jnp.clip(1.0 + jnp.sum(y * y, axis=1, keepdims=True), EPS, None))
    yn = jnp.maximum(_rownorm(y), MIN_NORM)
    yhat = y / yn
    alpha = jnp.sum(yhat * bb, axis=1, keepdims=True)
    w = bb - alpha * (1.0 - x0) * yhat
    first = jnp.sum(y * w, axis=1, keepdims=True) / jnp.clip(x0, EPS, None)
    md = jnp.sum(w * w, axis=1, keepdims=True) - first * first
    normu = jnp.minimum(jnp.sqrt(jnp.clip(md, EPS, None)), MAX_NORM)
    theta = jnp.maximum(normu, MIN_NORM)
    h = _cosh(theta) * y + _sinh(theta) * w / theta
    hn2 = jnp.sum(h * h, axis=1, keepdims=True)
    h0 = jnp.sqrt(jnp.clip(1.0 + hn2, EPS, None))
    y2n = jnp.maximum(jnp.sqrt(hn2), MIN_NORM)
    return _arcosh(jnp.clip(h0, 1.0 + EPS, None)) * h / y2n


def _layer1_body(x_ref, w1t_ref, b1_ref, out_ref):
    x = x_ref[...]
    sp = (lax.broadcasted_iota(jnp.int32, (1, D1), 1) != 0).astype(jnp.float32)
    x0 = jnp.sum(x * (1.0 - sp), axis=1, keepdims=True)
    y = x * sp
    yn = jnp.maximum(_rownorm(y), MIN_NORM)
    t0 = _arcosh(jnp.clip(x0, 1.0 + EPS, None)) * y / yn
    mu = jnp.dot(t0, w1t_ref[...], preferred_element_type=jnp.float32) * sp
    bb = b1_ref[...] * sp
    t = _hyp_linear_tangent(mu, bb)
    ones = jnp.ones((t.shape[0], DEG_W), jnp.float32)
    out_ref[...] = jnp.concatenate([t, ones], axis=1)


def _layer2_body(p_ref, w2t_ref, b2_ref, out_ref):
    s = p_ref[...]
    lane = lax.broadcasted_iota(jnp.int32, (1, DW1), 1)
    deg = jnp.sum(s * (lane == D1).astype(jnp.float32), axis=1, keepdims=True)
    a = s[:, :D1] / jnp.maximum(deg, 1.0)
    r = jnp.maximum(a, 0.0)
    sp = (lax.broadcasted_iota(jnp.int32, (1, D2), 1) != 0).astype(jnp.float32)
    mu = jnp.dot(r, w2t_ref[...], preferred_element_type=jnp.float32) * sp
    bb = b2_ref[...] * sp
    out_ref[...] = _hyp_linear_tangent(mu, bb)


def _pool_body(q_ref, dp_ref, batch_ref, out_ref):
    q = q_ref[...]
    s = q[0] + q[1]
    dsum = dp_ref[...]
    lane16 = lax.broadcasted_iota(jnp.int32, (1, DEG_W), 1)
    deg = jnp.sum(dsum * (lane16 == 0).astype(jnp.float32), axis=1, keepdims=True)
    a = s / jnp.maximum(deg, 1.0)
    b = batch_ref[...]
    gid = lax.broadcasted_iota(jnp.int32, (NG, N), 0)
    oh = (gid == b).astype(jnp.float32)
    gsum = jnp.dot(oh, a, preferred_element_type=jnp.float32)
    cnt = jnp.sum(oh, axis=1, keepdims=True)
    g = gsum / jnp.maximum(cnt, 1.0)
    sh = g - jnp.max(g, axis=1, keepdims=True)
    out_ref[...] = sh - jnp.log(jnp.sum(jnp.exp(sh), axis=1, keepdims=True))


NBUF = 4
HW1 = DW1 // 2


def _ring(gather, scatter, nch):
    for b in range(min(NBUF - 1, nch)):
        gather(b, b).start()

    def body(p, carry):
        for q in range(NBUF):
            j = p * NBUF + q
            gather(j, q).wait()
            scatter(j, q).start(add=True)
            bn = (q + NBUF - 1) % NBUF

            @pl.when(j >= 1)
            def _():
                scatter(j - 1, bn).wait()

            @pl.when(j + NBUF - 1 < nch)
            def _():
                gather(j + NBUF - 1, bn).start()
        return carry

    lax.fori_loop(0, nch // NBUF, body, 0)
    for j in range(nch - nch % NBUF, nch):
        b = j % NBUF
        gather(j, b).wait()
        scatter(j, b).start(add=True)
        if j >= 1:
            scatter(j - 1, (b + NBUF - 1) % NBUF).wait()
        if j + NBUF - 1 < nch:
            gather(j + NBUF - 1, (b + NBUF - 1) % NBUF).start()
    scatter(nch - 1, (nch - 1) % NBUF).wait()


def _sc_colsplit_agg():
    nch = E // NSUB // K
    mesh = plsc.VectorSubcoreMesh(core_axis_name="c", subcore_axis_name="s")

    @functools.partial(
        pl.kernel,
        out_type=jax.ShapeDtypeStruct((N, DW1), jnp.float32),
        mesh=mesh,
        scratch_types=[
            pltpu.VMEM((nch, K), jnp.int32),
            pltpu.VMEM((nch, K), jnp.int32),
            [pltpu.VMEM((K, HW1), jnp.float32)] * NBUF,
            [pltpu.SemaphoreType.DMA] * NBUF,
            [pltpu.SemaphoreType.DMA] * NBUF,
            pltpu.VMEM_SHARED((N, HW1), jnp.float32),
        ],
        compiler_params=pltpu.CompilerParams(use_tc_tiling_on_sc=False),
    )
    def agg(table, src, dst, zeros, out, src_sl, dst_sl, rows, gsem, ssem, acc):
        c = lax.axis_index("c")
        s = lax.axis_index("s")
        rb = s * ROWS_PER_TILE
        pltpu.sync_copy(zeros.at[pl.ds(rb, ROWS_PER_TILE)],
                        acc.at[pl.ds(rb, ROWS_PER_TILE)])
        pltpu.sync_copy(src.at[pl.ds(c * (E // K) + s * nch, nch)], src_sl)
        pltpu.sync_copy(dst.at[pl.ds(s * nch, nch)], dst_sl)
        plsc.subcore_barrier()

        def gather(j, b):
            return pltpu.make_async_copy(table.at[src_sl.at[j]], rows[b],
                                         gsem[b])

        def scatter(j, b):
            return pltpu.make_async_copy(rows[b], acc.at[dst_sl.at[j]],
                                         ssem[b])

        _ring(gather, scatter, nch)
        plsc.subcore_barrier()
        pltpu.sync_copy(acc.at[pl.ds(rb, ROWS_PER_TILE)],
                        out.at[pl.ds(rb, ROWS_PER_TILE), pl.ds(c * HW1, HW1)])

    return agg


def _sc_edgesplit_agg(width):
    nch = EDGES_PER_TILE // K
    mesh = plsc.VectorSubcoreMesh(core_axis_name="c", subcore_axis_name="s")

    @functools.partial(
        pl.kernel,
        out_type=jax.ShapeDtypeStruct((NCORES * N, width), jnp.float32),
        mesh=mesh,
        scratch_types=[
            pltpu.VMEM((nch, K), jnp.int32),
            pltpu.VMEM((nch, K), jnp.int32),
            [pltpu.VMEM((K, width), jnp.float32)] * NBUF,
            [pltpu.SemaphoreType.DMA] * NBUF,
            [pltpu.SemaphoreType.DMA] * NBUF,
            pltpu.VMEM_SHARED((N, width), jnp.float32),
        ],
        compiler_params=pltpu.CompilerParams(use_tc_tiling_on_sc=False),
    )
    def agg(table, src, dst, zeros, out, src_sl, dst_sl, rows, gsem, ssem, acc):
        c = lax.axis_index("c")
        s = lax.axis_index("s")
        rb = s * ROWS_PER_TILE
        pltpu.sync_copy(zeros.at[pl.ds(rb, ROWS_PER_TILE)],
                        acc.at[pl.ds(rb, ROWS_PER_TILE)])
        cb = (c * NSUB + s) * nch
        pltpu.sync_copy(src.at[pl.ds(cb, nch)], src_sl)
        pltpu.sync_copy(dst.at[pl.ds(cb, nch)], dst_sl)
        plsc.subcore_barrier()

        def gather(j, b):
            return pltpu.make_async_copy(table.at[src_sl.at[j]], rows[b],
                                         gsem[b])

        def scatter(j, b):
            return pltpu.make_async_copy(rows[b], acc.at[dst_sl.at[j]],
                                         ssem[b])

        _ring(gather, scatter, nch)
        plsc.subcore_barrier()
        pltpu.sync_copy(acc.at[pl.ds(rb, ROWS_PER_TILE)],
                        out.at[pl.ds(c * N + rb, ROWS_PER_TILE)])

    return agg


_sc_colsplit_agg = functools.lru_cache(maxsize=None)(_sc_colsplit_agg)
_sc_edgesplit_agg = functools.lru_cache(maxsize=None)(_sc_edgesplit_agg)


def kernel(x, edge_index, batch, W1, b1, W2, b2):
    src = edge_index[0]
    dst2 = edge_index[1].reshape(E // K, K)
    srcb = jnp.stack([src * 2, src * 2 + 1]).reshape(2 * (E // K), K)
    src2 = src.reshape(E // K, K)

    t1 = pl.pallas_call(
        _layer1_body,
        grid=(N // BLK,),
        in_specs=[
            pl.BlockSpec((BLK, D1), lambda i: (i, 0)),
            pl.BlockSpec((D1, D1), lambda i: (0, 0)),
            pl.BlockSpec((1, D1), lambda i: (0, 0)),
        ],
        out_specs=pl.BlockSpec((BLK, DW1), lambda i: (i, 0)),
        out_shape=jax.ShapeDtypeStruct((N, DW1), jnp.float32),
    )(x, W1.T, b1.reshape(1, -1))

    a1 = _sc_colsplit_agg()(t1.reshape(2 * N, HW1), srcb, dst2,
                            jnp.zeros((N, HW1), jnp.float32))

    t2 = pl.pallas_call(
        _layer2_body,
        grid=(N // BLK,),
        in_specs=[
            pl.BlockSpec((BLK, DW1), lambda i: (i, 0)),
            pl.BlockSpec((D1, D2), lambda i: (0, 0)),
            pl.BlockSpec((1, D2), lambda i: (0, 0)),
        ],
        out_specs=pl.BlockSpec((BLK, D2), lambda i: (i, 0)),
        out_shape=jax.ShapeDtypeStruct((N, D2), jnp.float32),
    )(a1, W2.T, b2.reshape(1, -1))

    p2 = _sc_edgesplit_agg(D2)(t2, src2, dst2,
                               jnp.zeros((N, D2), jnp.float32))
    p2 = p2.reshape(NCORES, N, D2)

    out = pl.pallas_call(
        _pool_body,
        in_specs=[
            pl.BlockSpec((NCORES, N, D2), lambda: (0, 0, 0)),
            pl.BlockSpec((N, DEG_W), lambda: (0, 0)),
            pl.BlockSpec((1, N), lambda: (0, 0)),
        ],
        out_specs=pl.BlockSpec((NG, D2), lambda: (0, 0)),
        out_shape=jax.ShapeDtypeStruct((NG, D2), jnp.float32),
    )(p2, a1[:, D1:], batch.reshape(1, -1))

    return out


# speedup vs baseline: 12.9780x; 1.0577x over previous
import functools

import jax
import jax.numpy as jnp
from jax import lax
from jax.experimental import pallas as pl
from jax.experimental.pallas import tpu as pltpu
from jax.experimental.pallas import tpu_sc as plsc

EPS = 1e-7
MIN_NORM = 1e-15
MAX_NORM = 1e6

N = 10000
E = 320000
D1 = 128
DEG_W = 16
DW1 = D1 + DEG_W
D2 = 64
NG = 128

NCORES = 2
NSUB = 16
EDGES_PER_SC = E // NCORES
EDGES_PER_TILE = E // (NCORES * NSUB)
K = 80
ROWS_PER_TILE = N // NSUB

BLK = 1000


def _sinh(t):
    e = jnp.exp(t)
    return 0.5 * (e - 1.0 / e)


def _cosh(t):
    e = jnp.exp(t)
    return 0.5 * (e + 1.0 / e)


def _arcosh(t):
    return jnp.log(t + jnp.sqrt(jnp.clip(t * t - 1.0, 1e-15, None)))


def _rownorm(v):
    return jnp.sqrt(jnp.sum(v * v, axis=1, keepdims=True))


def _hyp_linear_tangent(mu, bb):
    xn = jnp.maximum(_rownorm(mu), MIN_NORM)
    y = _sinh(xn) * mu / xn
    x0 = jnp.sqrt(jnp.clip(1.0 + jnp.sum(y * y, axis=1, keepdims=True), EPS, None))
    yn = jnp.maximum(_rownorm(y), MIN_NORM)
    yhat = y / yn
    alpha = jnp.sum(yhat * bb, axis=1, keepdims=True)
    w = bb - alpha * (1.0 - x0) * yhat
    first = jnp.sum(y * w, axis=1, keepdims=True) / jnp.clip(x0, EPS, None)
    md = jnp.sum(w * w, axis=1, keepdims=True) - first * first
    normu = jnp.minimum(jnp.sqrt(jnp.clip(md, EPS, None)), MAX_NORM)
    theta = jnp.maximum(normu, MIN_NORM)
    e = jnp.exp(theta)
    h = 0.5 * (e + 1.0 / e) * y + 0.5 * (e - 1.0 / e) * w / theta
    hn2 = jnp.sum(h * h, axis=1, keepdims=True)
    h0 = jnp.sqrt(jnp.clip(1.0 + hn2, EPS, None))
    y2n = jnp.maximum(jnp.sqrt(hn2), MIN_NORM)
    return _arcosh(jnp.clip(h0, 1.0 + EPS, None)) * h / y2n


def _layer1_body(x_ref, w1t_ref, b1_ref, out_ref):
    x = x_ref[...]
    sp = (lax.broadcasted_iota(jnp.int32, (1, D1), 1) != 0).astype(jnp.float32)
    x0 = jnp.sum(x * (1.0 - sp), axis=1, keepdims=True)
    y = x * sp
    yn = jnp.maximum(_rownorm(y), MIN_NORM)
    t0 = _arcosh(jnp.clip(x0, 1.0 + EPS, None)) * y / yn
    mu = jnp.dot(t0, w1t_ref[...], preferred_element_type=jnp.float32) * sp
    bb = b1_ref[...] * sp
    t = _hyp_linear_tangent(mu, bb)
    ones = jnp.ones((t.shape[0], DEG_W), jnp.float32)
    out_ref[...] = jnp.concatenate([t, ones], axis=1)


def _layer2_body(p_ref, w2t_ref, b2_ref, out_ref):
    s = p_ref[...]
    lane = lax.broadcasted_iota(jnp.int32, (1, DW1), 1)
    deg = jnp.sum(s * (lane == D1).astype(jnp.float32), axis=1, keepdims=True)
    a = s[:, :D1] / jnp.maximum(deg, 1.0)
    r = jnp.maximum(a, 0.0)
    sp = (lax.broadcasted_iota(jnp.int32, (1, D2), 1) != 0).astype(jnp.float32)
    mu = jnp.dot(r, w2t_ref[...], preferred_element_type=jnp.float32) * sp
    bb = b2_ref[...] * sp
    out_ref[...] = _hyp_linear_tangent(mu, bb)


def _pool_body(q_ref, dp_ref, batch_ref, out_ref):
    q = q_ref[...]
    s = q[0] + q[1]
    dsum = dp_ref[...]
    lane = lax.broadcasted_iota(jnp.int32, (1, DW1), 1)
    deg = jnp.sum(dsum * (lane == D1).astype(jnp.float32), axis=1, keepdims=True)
    a = s / jnp.maximum(deg, 1.0)
    b = batch_ref[...]
    gid = lax.broadcasted_iota(jnp.int32, (NG, N), 0)
    oh = (gid == b).astype(jnp.float32)
    gsum = jnp.dot(oh, a, preferred_element_type=jnp.float32)
    cnt = jnp.sum(oh, axis=1, keepdims=True)
    g = gsum / jnp.maximum(cnt, 1.0)
    sh = g - jnp.max(g, axis=1, keepdims=True)
    out_ref[...] = sh - jnp.log(jnp.sum(jnp.exp(sh), axis=1, keepdims=True))


HW1 = DW1 // 2


def _ring(gather, scatter, nch, nbuf):
    for b in range(min(nbuf - 1, nch)):
        gather(b, b).start()

    def body(p, carry):
        for q in range(nbuf):
            j = p * nbuf + q
            gather(j, q).wait()
            scatter(j, q).start(add=True)
            bn = (q + nbuf - 1) % nbuf

            @pl.when(j >= 1)
            def _():
                scatter(j - 1, bn).wait()

            @pl.when(j + nbuf - 1 < nch)
            def _():
                gather(j + nbuf - 1, bn).start()
        return carry

    lax.fori_loop(0, nch // nbuf, body, 0)
    for j in range(nch - nch % nbuf, nch):
        b = j % nbuf
        gather(j, b).wait()
        scatter(j, b).start(add=True)
        if j >= 1:
            scatter(j - 1, (b + nbuf - 1) % nbuf).wait()
        if j + nbuf - 1 < nch:
            gather(j + nbuf - 1, (b + nbuf - 1) % nbuf).start()
    scatter(nch - 1, (nch - 1) % nbuf).wait()


def _sc_colsplit_agg():
    nch = E // NSUB // K
    nbuf = 6
    mesh = plsc.VectorSubcoreMesh(core_axis_name="c", subcore_axis_name="s")

    @functools.partial(
        pl.kernel,
        out_type=jax.ShapeDtypeStruct((N, DW1), jnp.float32),
        mesh=mesh,
        scratch_types=[
            pltpu.VMEM((nch, K), jnp.int32),
            pltpu.VMEM((nch, K), jnp.int32),
            [pltpu.VMEM((K, HW1), jnp.float32)] * nbuf,
            [pltpu.SemaphoreType.DMA] * nbuf,
            [pltpu.SemaphoreType.DMA] * nbuf,
            pltpu.VMEM_SHARED((N, HW1), jnp.float32),
        ],
        compiler_params=pltpu.CompilerParams(use_tc_tiling_on_sc=False),
    )
    def agg(table, src, dst, zeros, out, src_sl, dst_sl, rows, gsem, ssem, acc):
        c = lax.axis_index("c")
        s = lax.axis_index("s")
        rb = s * ROWS_PER_TILE
        pltpu.sync_copy(zeros.at[pl.ds(rb, ROWS_PER_TILE)],
                        acc.at[pl.ds(rb, ROWS_PER_TILE)])
        pltpu.sync_copy(src.at[pl.ds(c * (E // K) + s * nch, nch)], src_sl)
        pltpu.sync_copy(dst.at[pl.ds(s * nch, nch)], dst_sl)
        plsc.subcore_barrier()

        def gather(j, b):
            return pltpu.make_async_copy(table.at[src_sl.at[j]], rows[b],
                                         gsem[b])

        def scatter(j, b):
            return pltpu.make_async_copy(rows[b], acc.at[dst_sl.at[j]],
                                         ssem[b])

        _ring(gather, scatter, nch, nbuf)
        plsc.subcore_barrier()
        pltpu.sync_copy(acc.at[pl.ds(rb, ROWS_PER_TILE)],
                        out.at[pl.ds(rb, ROWS_PER_TILE), pl.ds(c * HW1, HW1)])

    return agg


def _sc_edgesplit_agg(width):
    nch = EDGES_PER_TILE // K
    nbuf = 8
    mesh = plsc.VectorSubcoreMesh(core_axis_name="c", subcore_axis_name="s")

    @functools.partial(
        pl.kernel,
        out_type=jax.ShapeDtypeStruct((NCORES * N, width), jnp.float32),
        mesh=mesh,
        scratch_types=[
            pltpu.VMEM((nch, K), jnp.int32),
            pltpu.VMEM((nch, K), jnp.int32),
            [pltpu.VMEM((K, width), jnp.float32)] * nbuf,
            [pltpu.SemaphoreType.DMA] * nbuf,
            [pltpu.SemaphoreType.DMA] * nbuf,
            pltpu.VMEM_SHARED((N, width), jnp.float32),
        ],
        compiler_params=pltpu.CompilerParams(use_tc_tiling_on_sc=False),
    )
    def agg(table, src, dst, zeros, out, src_sl, dst_sl, rows, gsem, ssem, acc):
        c = lax.axis_index("c")
        s = lax.axis_index("s")
        rb = s * ROWS_PER_TILE
        pltpu.sync_copy(zeros.at[pl.ds(rb, ROWS_PER_TILE), pl.ds(0, width)],
                        acc.at[pl.ds(rb, ROWS_PER_TILE)])
        cb = (c * NSUB + s) * nch
        pltpu.sync_copy(src.at[pl.ds(cb, nch)], src_sl)
        pltpu.sync_copy(dst.at[pl.ds(cb, nch)], dst_sl)
        plsc.subcore_barrier()

        def gather(j, b):
            return pltpu.make_async_copy(table.at[src_sl.at[j]], rows[b],
                                         gsem[b])

        def scatter(j, b):
            return pltpu.make_async_copy(rows[b], acc.at[dst_sl.at[j]],
                                         ssem[b])

        _ring(gather, scatter, nch, nbuf)
        plsc.subcore_barrier()
        pltpu.sync_copy(acc.at[pl.ds(rb, ROWS_PER_TILE)],
                        out.at[pl.ds(c * N + rb, ROWS_PER_TILE)])

    return agg


_sc_colsplit_agg = functools.lru_cache(maxsize=None)(_sc_colsplit_agg)
_sc_edgesplit_agg = functools.lru_cache(maxsize=None)(_sc_edgesplit_agg)


def kernel(x, edge_index, batch, W1, b1, W2, b2):
    src = edge_index[0]
    dst2 = edge_index[1].reshape(E // K, K)
    srcb = jnp.stack([src * 2, src * 2 + 1]).reshape(2 * (E // K), K)
    src2 = src.reshape(E // K, K)

    t1 = pl.pallas_call(
        _layer1_body,
        grid=(N // BLK,),
        in_specs=[
            pl.BlockSpec((BLK, D1), lambda i: (i, 0)),
            pl.BlockSpec((D1, D1), lambda i: (0, 0)),
            pl.BlockSpec((1, D1), lambda i: (0, 0)),
        ],
        out_specs=pl.BlockSpec((BLK, DW1), lambda i: (i, 0)),
        out_shape=jax.ShapeDtypeStruct((N, DW1), jnp.float32),
    )(x, W1.T, b1.reshape(1, -1))

    zeros = jnp.zeros((N, HW1), jnp.float32)
    a1 = _sc_colsplit_agg()(t1.reshape(2 * N, HW1), srcb, dst2, zeros)

    t2 = pl.pallas_call(
        _layer2_body,
        grid=(N // BLK,),
        in_specs=[
            pl.BlockSpec((BLK, DW1), lambda i: (i, 0)),
            pl.BlockSpec((D1, D2), lambda i: (0, 0)),
            pl.BlockSpec((1, D2), lambda i: (0, 0)),
        ],
        out_specs=pl.BlockSpec((BLK, D2), lambda i: (i, 0)),
        out_shape=jax.ShapeDtypeStruct((N, D2), jnp.float32),
    )(a1, W2.T, b2.reshape(1, -1))

    p2 = _sc_edgesplit_agg(D2)(t2, src2, dst2, zeros)
    p2 = p2.reshape(NCORES, N, D2)

    out = pl.pallas_call(
        _pool_body,
        in_specs=[
            pl.BlockSpec((NCORES, N, D2), lambda: (0, 0, 0)),
            pl.BlockSpec((N, DW1), lambda: (0, 0)),
            pl.BlockSpec((1, N), lambda: (0, 0)),
        ],
        out_specs=pl.BlockSpec((NG, D2), lambda: (0, 0)),
        out_shape=jax.ShapeDtypeStruct((NG, D2), jnp.float32),
    )(p2, a1, batch.reshape(1, -1))

    return out


# speedup vs baseline: 14.9568x; 1.1525x over previous
import functools

import jax
import jax.numpy as jnp
from jax import lax
from jax.experimental import pallas as pl
from jax.experimental.pallas import tpu as pltpu
from jax.experimental.pallas import tpu_sc as plsc

EPS = 1e-7
MIN_NORM = 1e-15
MAX_NORM = 1e6

N = 10000
E = 320000
D1 = 128
D2 = 64
NG = 128

NCORES = 2
NSUB = 16
EDGES_PER_TILE = E // (NCORES * NSUB)
K = 80
ROWS_PER_TILE = N // NSUB
HW1 = D1 // 2

BLK = 1000


def _arcosh(t):
    return jnp.log(t + jnp.sqrt(jnp.clip(t * t - 1.0, 1e-15, None)))


def _hyp_linear_tangent(mu, bb):
    xn = jnp.maximum(jnp.sqrt(jnp.sum(mu * mu, axis=1, keepdims=True)),
                     MIN_NORM)
    ex = jnp.exp(xn)
    y = (0.5 * (ex - 1.0 / ex)) * mu / xn
    s2 = jnp.sum(y * y, axis=1, keepdims=True)
    x0 = jnp.sqrt(jnp.clip(1.0 + s2, EPS, None))
    yn = jnp.maximum(jnp.sqrt(s2), MIN_NORM)
    alpha = jnp.sum(y * bb, axis=1, keepdims=True) / yn
    w = bb - (alpha * (1.0 - x0) / yn) * y
    first = jnp.sum(y * w, axis=1, keepdims=True) / jnp.clip(x0, EPS, None)
    md = jnp.sum(w * w, axis=1, keepdims=True) - first * first
    normu = jnp.minimum(jnp.sqrt(jnp.clip(md, EPS, None)), MAX_NORM)
    theta = jnp.maximum(normu, MIN_NORM)
    e = jnp.exp(theta)
    h = 0.5 * (e + 1.0 / e) * y + (0.5 * (e - 1.0 / e) / theta) * w
    hn2 = jnp.sum(h * h, axis=1, keepdims=True)
    h0 = jnp.sqrt(jnp.clip(1.0 + hn2, EPS, None))
    y2n = jnp.maximum(jnp.sqrt(hn2), MIN_NORM)
    return (_arcosh(jnp.clip(h0, 1.0 + EPS, None)) / y2n) * h


def _layer1_body(x_ref, w1t_ref, b1_ref, out_ref):
    x = x_ref[...]
    sp = (lax.broadcasted_iota(jnp.int32, (1, D1), 1) != 0).astype(jnp.float32)
    x0 = jnp.sum(x * (1.0 - sp), axis=1, keepdims=True)
    y = x * sp
    yn = jnp.maximum(jnp.sqrt(jnp.sum(y * y, axis=1, keepdims=True)), MIN_NORM)
    t0 = (_arcosh(jnp.clip(x0, 1.0 + EPS, None)) / yn) * y
    mu = jnp.dot(t0, w1t_ref[...], preferred_element_type=jnp.float32) * sp
    bb = b1_ref[...] * sp
    t = _hyp_linear_tangent(mu, bb)
    out_ref[...] = t + (1.0 - sp)


def _layer2_body(p_ref, w2t_ref, b2_ref, out_ref):
    s = p_ref[...]
    sp1 = (lax.broadcasted_iota(jnp.int32, (1, D1), 1) != 0).astype(jnp.float32)
    deg = jnp.sum(s * (1.0 - sp1), axis=1, keepdims=True)
    a = s / jnp.maximum(deg, 1.0)
    r = jnp.maximum(a, 0.0) * sp1
    sp = (lax.broadcasted_iota(jnp.int32, (1, D2), 1) != 0).astype(jnp.float32)
    mu = jnp.dot(r, w2t_ref[...], preferred_element_type=jnp.float32) * sp
    bb = b2_ref[...] * sp
    out_ref[...] = _hyp_linear_tangent(mu, bb)


def _pool_body(q_ref, dp_ref, batch_ref, out_ref):
    q = q_ref[...]
    s = q[0] + q[1]
    dsum = dp_ref[...]
    lane = lax.broadcasted_iota(jnp.int32, (1, D1), 1)
    deg = jnp.sum(dsum * (lane == 0).astype(jnp.float32), axis=1, keepdims=True)
    a = s / jnp.maximum(deg, 1.0)
    b = batch_ref[...]
    gid = lax.broadcasted_iota(jnp.int32, (NG, N), 0)
    oh = (gid == b).astype(jnp.float32)
    gsum = jnp.dot(oh, a, preferred_element_type=jnp.float32)
    cnt = jnp.sum(oh, axis=1, keepdims=True)
    g = gsum / jnp.maximum(cnt, 1.0)
    sh = g - jnp.max(g, axis=1, keepdims=True)
    out_ref[...] = sh - jnp.log(jnp.sum(jnp.exp(sh), axis=1, keepdims=True))


def _ring(g_start, g_wait, s_start, s_wait, nch, nbuf):
    for b in range(min(nbuf - 1, nch)):
        g_start(b, b)

    def body(p, carry):
        for q in range(nbuf):
            j = p * nbuf + q
            g_wait(j, q)
            s_start(j, q)
            bn = (q + nbuf - 1) % nbuf

            @pl.when(j >= 1)
            def _():
                s_wait(j - 1, bn)

            @pl.when(j + nbuf - 1 < nch)
            def _():
                g_start(j + nbuf - 1, bn)
        return carry

    lax.fori_loop(0, nch // nbuf, body, 0)
    for j in range(nch - nch % nbuf, nch):
        b = j % nbuf
        g_wait(j, b)
        s_start(j, b)
        if j >= 1:
            s_wait(j - 1, (b + nbuf - 1) % nbuf)
        if j + nbuf - 1 < nch:
            g_start(j + nbuf - 1, (b + nbuf - 1) % nbuf)
    s_wait(nch - 1, (nch - 1) % nbuf)


def _sc_ring_agg(table, src, dst, zeros, out, src_sl, dst_sl, rows, gsem,
                 ssem, acc, *, src_base, dst_base, nch, nbuf):
    rb = lax.axis_index("s") * ROWS_PER_TILE
    pltpu.sync_copy(zeros.at[pl.ds(rb, ROWS_PER_TILE)],
                    acc.at[pl.ds(rb, ROWS_PER_TILE)])
    pltpu.sync_copy(src.at[pl.ds(src_base, nch)], src_sl)
    pltpu.sync_copy(dst.at[pl.ds(dst_base, nch)], dst_sl)
    plsc.subcore_barrier()

    def g_start(j, b):
        pltpu.make_async_copy(table.at[src_sl.at[j]], rows[b], gsem[b]).start()

    def g_wait(j, b):
        pltpu.make_async_copy(table.at[src_sl.at[j]], rows[b], gsem[b]).wait()

    def s_start(j, b):
        pltpu.make_async_copy(rows[b], acc.at[dst_sl.at[j]],
                              ssem[b]).start(add=True)

    def s_wait(j, b):
        pltpu.make_async_copy(rows[b], acc.at[dst_sl.at[j]], ssem[b]).wait()

    _ring(g_start, g_wait, s_start, s_wait, nch, nbuf)
    plsc.subcore_barrier()
    return rb


def _sc_colsplit_agg():
    nch = E // NSUB // K
    nbuf = 6
    mesh = plsc.VectorSubcoreMesh(core_axis_name="c", subcore_axis_name="s")

    @functools.partial(
        pl.kernel,
        out_type=jax.ShapeDtypeStruct((N, D1), jnp.float32),
        mesh=mesh,
        scratch_types=[
            pltpu.VMEM((nch, K), jnp.int32),
            pltpu.VMEM((nch, K), jnp.int32),
            [pltpu.VMEM((K, HW1), jnp.float32)] * nbuf,
            [pltpu.SemaphoreType.DMA] * nbuf,
            [pltpu.SemaphoreType.DMA] * nbuf,
            pltpu.VMEM_SHARED((N, HW1), jnp.float32),
        ],
        compiler_params=pltpu.CompilerParams(use_tc_tiling_on_sc=False),
    )
    def agg(table, src, dst, zeros, out, src_sl, dst_sl, rows, gsem, ssem,
            acc):
        c = lax.axis_index("c")
        s = lax.axis_index("s")
        rb = _sc_ring_agg(
            table, src, dst, zeros, out, src_sl, dst_sl, rows, gsem, ssem,
            acc, src_base=c * (E // K) + s * nch, dst_base=s * nch, nch=nch,
            nbuf=nbuf)
        pltpu.sync_copy(acc.at[pl.ds(rb, ROWS_PER_TILE)],
                        out.at[pl.ds(rb, ROWS_PER_TILE), pl.ds(c * HW1, HW1)])

    return agg


def _sc_edgesplit_agg(width):
    nch = EDGES_PER_TILE // K
    nbuf = 8
    mesh = plsc.VectorSubcoreMesh(core_axis_name="c", subcore_axis_name="s")

    @functools.partial(
        pl.kernel,
        out_type=jax.ShapeDtypeStruct((NCORES * N, width), jnp.float32),
        mesh=mesh,
        scratch_types=[
            pltpu.VMEM((nch, K), jnp.int32),
            pltpu.VMEM((nch, K), jnp.int32),
            [pltpu.VMEM((K, width), jnp.float32)] * nbuf,
            [pltpu.SemaphoreType.DMA] * nbuf,
            [pltpu.SemaphoreType.DMA] * nbuf,
            pltpu.VMEM_SHARED((N, width), jnp.float32),
        ],
        compiler_params=pltpu.CompilerParams(use_tc_tiling_on_sc=False),
    )
    def agg(table, src, dst, zeros, out, src_sl, dst_sl, rows, gsem, ssem,
            acc):
        c = lax.axis_index("c")
        s = lax.axis_index("s")
        cb = (c * NSUB + s) * nch
        rb = _sc_ring_agg(
            table, src, dst, zeros, out, src_sl, dst_sl, rows, gsem, ssem,
            acc, src_base=cb, dst_base=cb, nch=nch, nbuf=nbuf)
        pltpu.sync_copy(acc.at[pl.ds(rb, ROWS_PER_TILE)],
                        out.at[pl.ds(c * N + rb, ROWS_PER_TILE)])

    return agg


_sc_colsplit_agg = functools.lru_cache(maxsize=None)(_sc_colsplit_agg)
_sc_edgesplit_agg = functools.lru_cache(maxsize=None)(_sc_edgesplit_agg)


def kernel(x, edge_index, batch, W1, b1, W2, b2):
    src = edge_index[0]
    dst2 = edge_index[1].reshape(E // K, K)
    srcb = jnp.stack([src * 2, src * 2 + 1]).reshape(2 * (E // K), K)
    src2 = src.reshape(E // K, K)

    t1 = pl.pallas_call(
        _layer1_body,
        grid=(N // BLK,),
        in_specs=[
            pl.BlockSpec((BLK, D1), lambda i: (i, 0)),
            pl.BlockSpec((D1, D1), lambda i: (0, 0)),
            pl.BlockSpec((1, D1), lambda i: (0, 0)),
        ],
        out_specs=pl.BlockSpec((BLK, D1), lambda i: (i, 0)),
        out_shape=jax.ShapeDtypeStruct((N, D1), jnp.float32),
    )(x, W1.T, b1.reshape(1, -1))

    zeros = jnp.zeros((N, HW1), jnp.float32)
    a1 = _sc_colsplit_agg()(t1.reshape(2 * N, HW1), srcb, dst2, zeros)

    t2 = pl.pallas_call(
        _layer2_body,
        grid=(N // BLK,),
        in_specs=[
            pl.BlockSpec((BLK, D1), lambda i: (i, 0)),
            pl.BlockSpec((D1, D2), lambda i: (0, 0)),
            pl.BlockSpec((1, D2), lambda i: (0, 0)),
        ],
        out_specs=pl.BlockSpec((BLK, D2), lambda i: (i, 0)),
        out_shape=jax.ShapeDtypeStruct((N, D2), jnp.float32),
    )(a1, W2.T, b2.reshape(1, -1))

    p2 = _sc_edgesplit_agg(D2)(t2, src2, dst2, zeros)
    p2 = p2.reshape(NCORES, N, D2)

    out = pl.pallas_call(
        _pool_body,
        in_specs=[
            pl.BlockSpec((NCORES, N, D2), lambda: (0, 0, 0)),
            pl.BlockSpec((N, D1), lambda: (0, 0)),
            pl.BlockSpec((1, N), lambda: (0, 0)),
        ],
        out_specs=pl.BlockSpec((NG, D2), lambda: (0, 0)),
        out_shape=jax.ShapeDtypeStruct((NG, D2), jnp.float32),
    )(p2, a1, batch.reshape(1, -1))

    return out


# speedup vs baseline: 14.9702x; 1.0009x over previous
import functools

import jax
import jax.numpy as jnp
from jax import lax
from jax.experimental import pallas as pl
from jax.experimental.pallas import tpu as pltpu
from jax.experimental.pallas import tpu_sc as plsc

EPS = 1e-7
MIN_NORM = 1e-15
MAX_NORM = 1e6

N = 10000
E = 320000
D1 = 128
D2 = 64
NG = 128

NCORES = 2
NSUB = 16
EDGES_PER_TILE = E // (NCORES * NSUB)
K = 80
ROWS_PER_TILE = N // NSUB
HW1 = D1 // 2

BLK = 1000


def _arcosh(t):
    return jnp.log(t + jnp.sqrt(jnp.clip(t * t - 1.0, 1e-15, None)))


def _hyp_linear_tangent(mu, bb):
    xn = jnp.maximum(jnp.sqrt(jnp.sum(mu * mu, axis=1, keepdims=True)),
                     MIN_NORM)
    ex = jnp.exp(xn)
    y = (0.5 * (ex - 1.0 / ex)) * mu / xn
    s2 = jnp.sum(y * y, axis=1, keepdims=True)
    x0 = jnp.sqrt(jnp.clip(1.0 + s2, EPS, None))
    yn = jnp.maximum(jnp.sqrt(s2), MIN_NORM)
    alpha = jnp.sum(y * bb, axis=1, keepdims=True) / yn
    w = bb - (alpha * (1.0 - x0) / yn) * y
    first = jnp.sum(y * w, axis=1, keepdims=True) / jnp.clip(x0, EPS, None)
    md = jnp.sum(w * w, axis=1, keepdims=True) - first * first
    normu = jnp.minimum(jnp.sqrt(jnp.clip(md, EPS, None)), MAX_NORM)
    theta = jnp.maximum(normu, MIN_NORM)
    e = jnp.exp(theta)
    h = 0.5 * (e + 1.0 / e) * y + (0.5 * (e - 1.0 / e) / theta) * w
    hn2 = jnp.sum(h * h, axis=1, keepdims=True)
    h0 = jnp.sqrt(jnp.clip(1.0 + hn2, EPS, None))
    y2n = jnp.maximum(jnp.sqrt(hn2), MIN_NORM)
    return (_arcosh(jnp.clip(h0, 1.0 + EPS, None)) / y2n) * h


def _layer1_body(x_ref, w1t_ref, b1_ref, out_ref):
    x = x_ref[...]
    sp = (lax.broadcasted_iota(jnp.int32, (1, D1), 1) != 0).astype(jnp.float32)
    x0 = jnp.sum(x * (1.0 - sp), axis=1, keepdims=True)
    y = x * sp
    yn = jnp.maximum(jnp.sqrt(jnp.sum(y * y, axis=1, keepdims=True)), MIN_NORM)
    t0 = (_arcosh(jnp.clip(x0, 1.0 + EPS, None)) / yn) * y
    mu = jnp.dot(t0, w1t_ref[...], preferred_element_type=jnp.float32) * sp
    bb = b1_ref[...] * sp
    t = _hyp_linear_tangent(mu, bb)
    out_ref[...] = t + (1.0 - sp)


def _layer2_body(p_ref, w2t_ref, b2_ref, out_ref):
    s = p_ref[...]
    sp1 = (lax.broadcasted_iota(jnp.int32, (1, D1), 1) != 0).astype(jnp.float32)
    deg = jnp.sum(s * (1.0 - sp1), axis=1, keepdims=True)
    a = s / jnp.maximum(deg, 1.0)
    r = jnp.maximum(a, 0.0) * sp1
    sp = (lax.broadcasted_iota(jnp.int32, (1, D2), 1) != 0).astype(jnp.float32)
    mu = jnp.dot(r, w2t_ref[...], preferred_element_type=jnp.float32) * sp
    bb = b2_ref[...] * sp
    out_ref[...] = _hyp_linear_tangent(mu, bb)


def _pool_body(q_ref, dp_ref, batch_ref, out_ref):
    q = q_ref[...]
    s = q[0] + q[1]
    dsum = dp_ref[...]
    lane = lax.broadcasted_iota(jnp.int32, (1, D1), 1)
    deg = jnp.sum(dsum * (lane == 0).astype(jnp.float32), axis=1, keepdims=True)
    a = s / jnp.maximum(deg, 1.0)
    b = batch_ref[...]
    gid = lax.broadcasted_iota(jnp.int32, (NG, N), 0)
    oh = (gid == b).astype(jnp.float32)
    gsum = jnp.dot(oh, a, preferred_element_type=jnp.float32)
    cnt = jnp.sum(oh, axis=1, keepdims=True)
    g = gsum / jnp.maximum(cnt, 1.0)
    sh = g - jnp.max(g, axis=1, keepdims=True)
    out_ref[...] = sh - jnp.log(jnp.sum(jnp.exp(sh), axis=1, keepdims=True))


def _ring(g_start, g_wait, s_start, s_wait, nch, nbuf):
    for b in range(min(nbuf - 1, nch)):
        g_start(b, b)

    def body(p, carry):
        for q in range(nbuf):
            j = p * nbuf + q
            g_wait(j, q)
            s_start(j, q)
            bn = (q + nbuf - 1) % nbuf

            @pl.when(j >= 1)
            def _():
                s_wait(j - 1, bn)

            @pl.when(j + nbuf - 1 < nch)
            def _():
                g_start(j + nbuf - 1, bn)
        return carry

    lax.fori_loop(0, nch // nbuf, body, 0)
    for j in range(nch - nch % nbuf, nch):
        b = j % nbuf
        g_wait(j, b)
        s_start(j, b)
        if j >= 1:
            s_wait(j - 1, (b + nbuf - 1) % nbuf)
        if j + nbuf - 1 < nch:
            g_start(j + nbuf - 1, (b + nbuf - 1) % nbuf)
    s_wait(nch - 1, (nch - 1) % nbuf)


def _sc_ring_agg(table, src, dst, zeros, out, src_sl, dst_sl, rows, gsem,
                 ssem, acc, *, src_base, dst_base, nch, nbuf):
    rb = lax.axis_index("s") * ROWS_PER_TILE
    pltpu.sync_copy(zeros.at[pl.ds(rb, ROWS_PER_TILE)],
                    acc.at[pl.ds(rb, ROWS_PER_TILE)])
    pltpu.sync_copy(src.at[pl.ds(src_base, nch)], src_sl)
    pltpu.sync_copy(dst.at[pl.ds(dst_base, nch)], dst_sl)
    plsc.subcore_barrier()

    def g_start(j, b):
        pltpu.make_async_copy(table.at[src_sl.at[j]], rows[b], gsem[b]).start()

    def g_wait(j, b):
        pltpu.make_async_copy(table.at[src_sl.at[j]], rows[b], gsem[b]).wait()

    def s_start(j, b):
        pltpu.make_async_copy(rows[b], acc.at[dst_sl.at[j]],
                              ssem[b]).start(add=True)

    def s_wait(j, b):
        pltpu.make_async_copy(rows[b], acc.at[dst_sl.at[j]], ssem[b]).wait()

    _ring(g_start, g_wait, s_start, s_wait, nch, nbuf)
    plsc.subcore_barrier()
    return rb


def _sc_colsplit_agg():
    nch = E // NSUB // K
    nbuf = 8
    mesh = plsc.VectorSubcoreMesh(core_axis_name="c", subcore_axis_name="s")

    @functools.partial(
        pl.kernel,
        out_type=jax.ShapeDtypeStruct((N, D1), jnp.float32),
        mesh=mesh,
        scratch_types=[
            pltpu.VMEM((nch, K), jnp.int32),
            pltpu.VMEM((nch, K), jnp.int32),
            [pltpu.VMEM((K, HW1), jnp.float32)] * nbuf,
            [pltpu.SemaphoreType.DMA] * nbuf,
            [pltpu.SemaphoreType.DMA] * nbuf,
            pltpu.VMEM_SHARED((N, HW1), jnp.float32),
        ],
        compiler_params=pltpu.CompilerParams(use_tc_tiling_on_sc=False),
    )
    def agg(table, src, dst, zeros, out, src_sl, dst_sl, rows, gsem, ssem,
            acc):
        c = lax.axis_index("c")
        s = lax.axis_index("s")
        rb = _sc_ring_agg(
            table, src, dst, zeros, out, src_sl, dst_sl, rows, gsem, ssem,
            acc, src_base=c * (E // K) + s * nch, dst_base=s * nch, nch=nch,
            nbuf=nbuf)
        pltpu.sync_copy(acc.at[pl.ds(rb, ROWS_PER_TILE)],
                        out.at[pl.ds(rb, ROWS_PER_TILE), pl.ds(c * HW1, HW1)])

    return agg


def _sc_edgesplit_agg(width):
    nch = EDGES_PER_TILE // K
    nbuf = 8
    mesh = plsc.VectorSubcoreMesh(core_axis_name="c", subcore_axis_name="s")

    @functools.partial(
        pl.kernel,
        out_type=jax.ShapeDtypeStruct((NCORES * N, width), jnp.float32),
        mesh=mesh,
        scratch_types=[
            pltpu.VMEM((nch, K), jnp.int32),
            pltpu.VMEM((nch, K), jnp.int32),
            [pltpu.VMEM((K, width), jnp.float32)] * nbuf,
            [pltpu.SemaphoreType.DMA] * nbuf,
            [pltpu.SemaphoreType.DMA] * nbuf,
            pltpu.VMEM_SHARED((N, width), jnp.float32),
        ],
        compiler_params=pltpu.CompilerParams(use_tc_tiling_on_sc=False),
    )
    def agg(table, src, dst, zeros, out, src_sl, dst_sl, rows, gsem, ssem,
            acc):
        c = lax.axis_index("c")
        s = lax.axis_index("s")
        cb = (c * NSUB + s) * nch
        rb = _sc_ring_agg(
            table, src, dst, zeros, out, src_sl, dst_sl, rows, gsem, ssem,
            acc, src_base=cb, dst_base=cb, nch=nch, nbuf=nbuf)
        pltpu.sync_copy(acc.at[pl.ds(rb, ROWS_PER_TILE)],
                        out.at[pl.ds(c * N + rb, ROWS_PER_TILE)])

    return agg


_sc_colsplit_agg = functools.lru_cache(maxsize=None)(_sc_colsplit_agg)
_sc_edgesplit_agg = functools.lru_cache(maxsize=None)(_sc_edgesplit_agg)


def kernel(x, edge_index, batch, W1, b1, W2, b2):
    src = edge_index[0]
    dst2 = edge_index[1].reshape(E // K, K)
    srcb = jnp.stack([src * 2, src * 2 + 1]).reshape(2 * (E // K), K)
    src2 = src.reshape(E // K, K)

    t1 = pl.pallas_call(
        _layer1_body,
        grid=(N // BLK,),
        in_specs=[
            pl.BlockSpec((BLK, D1), lambda i: (i, 0)),
            pl.BlockSpec((D1, D1), lambda i: (0, 0)),
            pl.BlockSpec((1, D1), lambda i: (0, 0)),
        ],
        out_specs=pl.BlockSpec((BLK, D1), lambda i: (i, 0)),
        out_shape=jax.ShapeDtypeStruct((N, D1), jnp.float32),
    )(x, W1.T, b1.reshape(1, -1))

    zeros = jnp.zeros((N, HW1), jnp.float32)
    a1 = _sc_colsplit_agg()(t1.reshape(2 * N, HW1), srcb, dst2, zeros)

    t2 = pl.pallas_call(
        _layer2_body,
        grid=(N // BLK,),
        in_specs=[
            pl.BlockSpec((BLK, D1), lambda i: (i, 0)),
            pl.BlockSpec((D1, D2), lambda i: (0, 0)),
            pl.BlockSpec((1, D2), lambda i: (0, 0)),
        ],
        out_specs=pl.BlockSpec((BLK, D2), lambda i: (i, 0)),
        out_shape=jax.ShapeDtypeStruct((N, D2), jnp.float32),
    )(a1, W2.T, b2.reshape(1, -1))

    p2 = _sc_edgesplit_agg(D2)(t2, src2, dst2, zeros)
    p2 = p2.reshape(NCORES, N, D2)

    out = pl.pallas_call(
        _pool_body,
        in_specs=[
            pl.BlockSpec((NCORES, N, D2), lambda: (0, 0, 0)),
            pl.BlockSpec((N, D1), lambda: (0, 0)),
            pl.BlockSpec((1, N), lambda: (0, 0)),
        ],
        out_specs=pl.BlockSpec((NG, D2), lambda: (0, 0)),
        out_shape=jax.ShapeDtypeStruct((NG, D2), jnp.float32),
    )(p2, a1, batch.reshape(1, -1))

    return out


# speedup vs baseline: 15.4187x; 1.0300x over previous
import functools

import jax
import jax.numpy as jnp
from jax import lax
from jax.experimental import pallas as pl
from jax.experimental.pallas import tpu as pltpu
from jax.experimental.pallas import tpu_sc as plsc

EPS = 1e-7
MIN_NORM = 1e-15
MAX_NORM = 1e6

N = 10000
E = 320000
D1 = 128
D2 = 64
NG = 128

NCORES = 2
NSUB = 16
EDGES_PER_TILE = E // (NCORES * NSUB)
K = 80
ROWS_PER_TILE = N // NSUB
HW1 = D1 // 2

BLK = 1000


def _arcosh(t):
    return jnp.log(t + jnp.sqrt(jnp.clip(t * t - 1.0, 1e-15, None)))


def _hyp_linear_tangent(mu, bb):
    xn = jnp.maximum(jnp.sqrt(jnp.sum(mu * mu, axis=1, keepdims=True)),
                     MIN_NORM)
    ex = jnp.exp(xn)
    y = (0.5 * (ex - 1.0 / ex)) * mu / xn
    s2 = jnp.sum(y * y, axis=1, keepdims=True)
    x0 = jnp.sqrt(jnp.clip(1.0 + s2, EPS, None))
    yn = jnp.maximum(jnp.sqrt(s2), MIN_NORM)
    alpha = jnp.sum(y * bb, axis=1, keepdims=True) / yn
    w = bb - (alpha * (1.0 - x0) / yn) * y
    first = jnp.sum(y * w, axis=1, keepdims=True) / jnp.clip(x0, EPS, None)
    md = jnp.sum(w * w, axis=1, keepdims=True) - first * first
    normu = jnp.minimum(jnp.sqrt(jnp.clip(md, EPS, None)), MAX_NORM)
    theta = jnp.maximum(normu, MIN_NORM)
    e = jnp.exp(theta)
    h = 0.5 * (e + 1.0 / e) * y + (0.5 * (e - 1.0 / e) / theta) * w
    hn2 = jnp.sum(h * h, axis=1, keepdims=True)
    h0 = jnp.sqrt(jnp.clip(1.0 + hn2, EPS, None))
    y2n = jnp.maximum(jnp.sqrt(hn2), MIN_NORM)
    return (_arcosh(jnp.clip(h0, 1.0 + EPS, None)) / y2n) * h


def _layer1_body(x_ref, w1t_ref, b1_ref, out_ref):
    x = x_ref[...]
    sp = (lax.broadcasted_iota(jnp.int32, (1, D1), 1) != 0).astype(jnp.float32)
    x0 = jnp.sum(x * (1.0 - sp), axis=1, keepdims=True)
    y = x * sp
    yn = jnp.maximum(jnp.sqrt(jnp.sum(y * y, axis=1, keepdims=True)), MIN_NORM)
    t0 = (_arcosh(jnp.clip(x0, 1.0 + EPS, None)) / yn) * y
    mu = jnp.dot(t0, w1t_ref[...], preferred_element_type=jnp.float32) * sp
    bb = b1_ref[...] * sp
    t = _hyp_linear_tangent(mu, bb)
    out_ref[...] = t + (1.0 - sp)


def _layer2_body(p_ref, w2t_ref, b2_ref, out_ref):
    s = p_ref[...]
    sp1 = (lax.broadcasted_iota(jnp.int32, (1, D1), 1) != 0).astype(jnp.float32)
    deg = jnp.sum(s * (1.0 - sp1), axis=1, keepdims=True)
    a = s / jnp.maximum(deg, 1.0)
    r = jnp.maximum(a, 0.0) * sp1
    sp = (lax.broadcasted_iota(jnp.int32, (1, D2), 1) != 0).astype(jnp.float32)
    mu = jnp.dot(r, w2t_ref[...], preferred_element_type=jnp.float32) * sp
    bb = b2_ref[...] * sp
    out_ref[...] = _hyp_linear_tangent(mu, bb)


def _pool_body(q_ref, dp_ref, batch_ref, out_ref):
    q = q_ref[...]
    s = q[0] + q[1]
    dsum = dp_ref[...]
    lane = lax.broadcasted_iota(jnp.int32, (1, D1), 1)
    deg = jnp.sum(dsum * (lane == 0).astype(jnp.float32), axis=1, keepdims=True)
    a = s / jnp.maximum(deg, 1.0)
    b = batch_ref[...]
    gid = lax.broadcasted_iota(jnp.int32, (NG, N), 0)
    oh = (gid == b).astype(jnp.float32)
    gsum = jnp.dot(oh, a, preferred_element_type=jnp.float32)
    cnt = jnp.sum(oh, axis=1, keepdims=True)
    g = gsum / jnp.maximum(cnt, 1.0)
    sh = g - jnp.max(g, axis=1, keepdims=True)
    out_ref[...] = sh - jnp.log(jnp.sum(jnp.exp(sh), axis=1, keepdims=True))


def _ring(g_start, g_wait, s_start, s_wait, nch, nbuf):
    for b in range(min(nbuf - 1, nch)):
        g_start(b, b)

    def body(p, carry):
        for q in range(nbuf):
            j = p * nbuf + q
            g_wait(j, q)
            s_start(j, q)
            bn = (q + nbuf - 1) % nbuf

            @pl.when(j >= 1)
            def _():
                s_wait(j - 1, bn)

            @pl.when(j + nbuf - 1 < nch)
            def _():
                g_start(j + nbuf - 1, bn)
        return carry

    lax.fori_loop(0, nch // nbuf, body, 0)
    for j in range(nch - nch % nbuf, nch):
        b = j % nbuf
        g_wait(j, b)
        s_start(j, b)
        if j >= 1:
            s_wait(j - 1, (b + nbuf - 1) % nbuf)
        if j + nbuf - 1 < nch:
            g_start(j + nbuf - 1, (b + nbuf - 1) % nbuf)
    s_wait(nch - 1, (nch - 1) % nbuf)


def _sc_ring_agg(table, src, dst, zeros, out, src_sl, dst_sl, rows, gsem,
                 ssem, acc, *, src_base, dst_base, nch, nbuf, src_xform=None):
    rb = lax.axis_index("s") * ROWS_PER_TILE
    pltpu.sync_copy(zeros.at[pl.ds(rb, ROWS_PER_TILE)],
                    acc.at[pl.ds(rb, ROWS_PER_TILE)])
    pltpu.sync_copy(src.at[pl.ds(src_base, nch)], src_sl)
    pltpu.sync_copy(dst.at[pl.ds(dst_base, nch)], dst_sl)
    if src_xform is not None:
        def xf(j, carry):
            for k in range(K // 16):
                idx = (j, pl.ds(16 * k, 16))
                src_sl[idx] = src_xform(src_sl[idx])
            return carry

        lax.fori_loop(0, nch, xf, 0)
    plsc.subcore_barrier()

    def g_start(j, b):
        pltpu.make_async_copy(table.at[src_sl.at[j]], rows[b], gsem[b]).start()

    def g_wait(j, b):
        pltpu.make_async_copy(table.at[src_sl.at[j]], rows[b], gsem[b]).wait()

    def s_start(j, b):
        pltpu.make_async_copy(rows[b], acc.at[dst_sl.at[j]],
                              ssem[b]).start(add=True)

    def s_wait(j, b):
        pltpu.make_async_copy(rows[b], acc.at[dst_sl.at[j]], ssem[b]).wait()

    _ring(g_start, g_wait, s_start, s_wait, nch, nbuf)
    plsc.subcore_barrier()
    return rb


def _sc_colsplit_agg():
    nch = E // NSUB // K
    nbuf = 8
    mesh = plsc.VectorSubcoreMesh(core_axis_name="c", subcore_axis_name="s")

    @functools.partial(
        pl.kernel,
        out_type=jax.ShapeDtypeStruct((N, D1), jnp.float32),
        mesh=mesh,
        scratch_types=[
            pltpu.VMEM((nch, K), jnp.int32),
            pltpu.VMEM((nch, K), jnp.int32),
            [pltpu.VMEM((K, HW1), jnp.float32)] * nbuf,
            [pltpu.SemaphoreType.DMA] * nbuf,
            [pltpu.SemaphoreType.DMA] * nbuf,
            pltpu.VMEM_SHARED((N, HW1), jnp.float32),
        ],
        compiler_params=pltpu.CompilerParams(use_tc_tiling_on_sc=False),
    )
    def agg(table, src, dst, zeros, out, src_sl, dst_sl, rows, gsem, ssem,
            acc):
        c = lax.axis_index("c")
        s = lax.axis_index("s")
        rb = _sc_ring_agg(
            table, src, dst, zeros, out, src_sl, dst_sl, rows, gsem, ssem,
            acc, src_base=s * nch, dst_base=s * nch, nch=nch, nbuf=nbuf,
            src_xform=lambda v: v * 2 + c)
        pltpu.sync_copy(acc.at[pl.ds(rb, ROWS_PER_TILE)],
                        out.at[pl.ds(rb, ROWS_PER_TILE), pl.ds(c * HW1, HW1)])

    return agg


def _sc_edgesplit_agg(width):
    nch = EDGES_PER_TILE // K
    nbuf = 8
    mesh = plsc.VectorSubcoreMesh(core_axis_name="c", subcore_axis_name="s")

    @functools.partial(
        pl.kernel,
        out_type=jax.ShapeDtypeStruct((NCORES * N, width), jnp.float32),
        mesh=mesh,
        scratch_types=[
            pltpu.VMEM((nch, K), jnp.int32),
            pltpu.VMEM((nch, K), jnp.int32),
            [pltpu.VMEM((K, width), jnp.float32)] * nbuf,
            [pltpu.SemaphoreType.DMA] * nbuf,
            [pltpu.SemaphoreType.DMA] * nbuf,
            pltpu.VMEM_SHARED((N, width), jnp.float32),
        ],
        compiler_params=pltpu.CompilerParams(use_tc_tiling_on_sc=False),
    )
    def agg(table, src, dst, zeros, out, src_sl, dst_sl, rows, gsem, ssem,
            acc):
        c = lax.axis_index("c")
        s = lax.axis_index("s")
        cb = (c * NSUB + s) * nch
        rb = _sc_ring_agg(
            table, src, dst, zeros, out, src_sl, dst_sl, rows, gsem, ssem,
            acc, src_base=cb, dst_base=cb, nch=nch, nbuf=nbuf)
        pltpu.sync_copy(acc.at[pl.ds(rb, ROWS_PER_TILE)],
                        out.at[pl.ds(c * N + rb, ROWS_PER_TILE)])

    return agg


_sc_colsplit_agg = functools.lru_cache(maxsize=None)(_sc_colsplit_agg)
_sc_edgesplit_agg = functools.lru_cache(maxsize=None)(_sc_edgesplit_agg)


def kernel(x, edge_index, batch, W1, b1, W2, b2):
    src2 = edge_index[0].reshape(E // K, K)
    dst2 = edge_index[1].reshape(E // K, K)

    t1 = pl.pallas_call(
        _layer1_body,
        grid=(N // BLK,),
        in_specs=[
            pl.BlockSpec((BLK, D1), lambda i: (i, 0)),
            pl.BlockSpec((D1, D1), lambda i: (0, 0)),
            pl.BlockSpec((1, D1), lambda i: (0, 0)),
        ],
        out_specs=pl.BlockSpec((BLK, D1), lambda i: (i, 0)),
        out_shape=jax.ShapeDtypeStruct((N, D1), jnp.float32),
    )(x, W1.T, b1.reshape(1, -1))

    zeros = jnp.zeros((N, HW1), jnp.float32)
    a1 = _sc_colsplit_agg()(t1.reshape(2 * N, HW1), src2, dst2, zeros)

    t2 = pl.pallas_call(
        _layer2_body,
        grid=(N // BLK,),
        in_specs=[
            pl.BlockSpec((BLK, D1), lambda i: (i, 0)),
            pl.BlockSpec((D1, D2), lambda i: (0, 0)),
            pl.BlockSpec((1, D2), lambda i: (0, 0)),
        ],
        out_specs=pl.BlockSpec((BLK, D2), lambda i: (i, 0)),
        out_shape=jax.ShapeDtypeStruct((N, D2), jnp.float32),
    )(a1, W2.T, b2.reshape(1, -1))

    p2 = _sc_edgesplit_agg(D2)(t2, src2, dst2, zeros)
    p2 = p2.reshape(NCORES, N, D2)

    out = pl.pallas_call(
        _pool_body,
        in_specs=[
            pl.BlockSpec((NCORES, N, D2), lambda: (0, 0, 0)),
            pl.BlockSpec((N, D1), lambda: (0, 0)),
            pl.BlockSpec((1, N), lambda: (0, 0)),
        ],
        out_specs=pl.BlockSpec((NG, D2), lambda: (0, 0)),
        out_shape=jax.ShapeDtypeStruct((NG, D2), jnp.float32),
    )(p2, a1, batch.reshape(1, -1))

    return out


# speedup vs baseline: 15.5363x; 1.0076x over previous
import functools

import jax
import jax.numpy as jnp
from jax import lax
from jax.experimental import pallas as pl
from jax.experimental.pallas import tpu as pltpu
from jax.experimental.pallas import tpu_sc as plsc

EPS = 1e-7
MIN_NORM = 1e-15
MAX_NORM = 1e6

N = 10000
E = 320000
D1 = 128
D2 = 64
NG = 128

NCORES = 2
NSUB = 16
EDGES_PER_TILE = E // (NCORES * NSUB)
K = 80
ROWS_PER_TILE = N // NSUB
HW1 = D1 // 2

BLK = 2000


def _arcosh(t):
    return jnp.log(t + jnp.sqrt(jnp.clip(t * t - 1.0, 1e-15, None)))


def _hyp_linear_tangent(mu, bb):
    xn = jnp.maximum(jnp.sqrt(jnp.sum(mu * mu, axis=1, keepdims=True)),
                     MIN_NORM)
    ex = jnp.exp(xn)
    y = (0.5 * (ex - 1.0 / ex)) * mu / xn
    s2 = jnp.sum(y * y, axis=1, keepdims=True)
    x0 = jnp.sqrt(jnp.clip(1.0 + s2, EPS, None))
    yn = jnp.maximum(jnp.sqrt(s2), MIN_NORM)
    alpha = jnp.sum(y * bb, axis=1, keepdims=True) / yn
    w = bb - (alpha * (1.0 - x0) / yn) * y
    first = jnp.sum(y * w, axis=1, keepdims=True) / jnp.clip(x0, EPS, None)
    md = jnp.sum(w * w, axis=1, keepdims=True) - first * first
    normu = jnp.minimum(jnp.sqrt(jnp.clip(md, EPS, None)), MAX_NORM)
    theta = jnp.maximum(normu, MIN_NORM)
    e = jnp.exp(theta)
    h = 0.5 * (e + 1.0 / e) * y + (0.5 * (e - 1.0 / e) / theta) * w
    hn2 = jnp.sum(h * h, axis=1, keepdims=True)
    h0 = jnp.sqrt(jnp.clip(1.0 + hn2, EPS, None))
    y2n = jnp.maximum(jnp.sqrt(hn2), MIN_NORM)
    return (_arcosh(jnp.clip(h0, 1.0 + EPS, None)) / y2n) * h


def _layer1_body(x_ref, w1t_ref, b1_ref, out_ref):
    x = x_ref[...]
    sp = (lax.broadcasted_iota(jnp.int32, (1, D1), 1) != 0).astype(jnp.float32)
    x0 = jnp.sum(x * (1.0 - sp), axis=1, keepdims=True)
    y = x * sp
    yn = jnp.maximum(jnp.sqrt(jnp.sum(y * y, axis=1, keepdims=True)), MIN_NORM)
    t0 = (_arcosh(jnp.clip(x0, 1.0 + EPS, None)) / yn) * y
    mu = jnp.dot(t0, w1t_ref[...], preferred_element_type=jnp.float32) * sp
    bb = b1_ref[...] * sp
    t = _hyp_linear_tangent(mu, bb)
    out_ref[...] = t + (1.0 - sp)


def _layer2_body(p_ref, w2t_ref, b2_ref, out_ref):
    s = p_ref[...]
    sp1 = (lax.broadcasted_iota(jnp.int32, (1, D1), 1) != 0).astype(jnp.float32)
    deg = jnp.sum(s * (1.0 - sp1), axis=1, keepdims=True)
    a = s / jnp.maximum(deg, 1.0)
    r = jnp.maximum(a, 0.0) * sp1
    sp = (lax.broadcasted_iota(jnp.int32, (1, D2), 1) != 0).astype(jnp.float32)
    mu = jnp.dot(r, w2t_ref[...], preferred_element_type=jnp.float32) * sp
    bb = b2_ref[...] * sp
    out_ref[...] = _hyp_linear_tangent(mu, bb)


def _pool_body(q_ref, dp_ref, batch_ref, out_ref):
    q = q_ref[...]
    s = q[0] + q[1]
    dsum = dp_ref[...]
    lane = lax.broadcasted_iota(jnp.int32, (1, D1), 1)
    deg = jnp.sum(dsum * (lane == 0).astype(jnp.float32), axis=1, keepdims=True)
    a = s / jnp.maximum(deg, 1.0)
    b = batch_ref[...]
    gid = lax.broadcasted_iota(jnp.int32, (NG, N), 0)
    oh = (gid == b).astype(jnp.float32)
    gsum = jnp.dot(oh, a, preferred_element_type=jnp.float32)
    cnt = jnp.sum(oh, axis=1, keepdims=True)
    g = gsum / jnp.maximum(cnt, 1.0)
    sh = g - jnp.max(g, axis=1, keepdims=True)
    out_ref[...] = sh - jnp.log(jnp.sum(jnp.exp(sh), axis=1, keepdims=True))


def _ring(g_start, g_wait, s_start, s_wait, nch, nbuf):
    for b in range(min(nbuf - 1, nch)):
        g_start(b, b)

    def body(p, carry):
        for q in range(nbuf):
            j = p * nbuf + q
            g_wait(j, q)
            s_start(j, q)
            bn = (q + nbuf - 1) % nbuf

            @pl.when(j >= 1)
            def _():
                s_wait(j - 1, bn)

            @pl.when(j + nbuf - 1 < nch)
            def _():
                g_start(j + nbuf - 1, bn)
        return carry

    lax.fori_loop(0, nch // nbuf, body, 0)
    for j in range(nch - nch % nbuf, nch):
        b = j % nbuf
        g_wait(j, b)
        s_start(j, b)
        if j >= 1:
            s_wait(j - 1, (b + nbuf - 1) % nbuf)
        if j + nbuf - 1 < nch:
            g_start(j + nbuf - 1, (b + nbuf - 1) % nbuf)
    s_wait(nch - 1, (nch - 1) % nbuf)


def _sc_ring_agg(table, src, dst, zeros, out, src_sl, dst_sl, rows, gsem,
                 ssem, acc, *, src_base, dst_base, nch, nbuf, src_xform=None):
    rb = lax.axis_index("s") * ROWS_PER_TILE
    pltpu.sync_copy(zeros.at[pl.ds(rb, ROWS_PER_TILE)],
                    acc.at[pl.ds(rb, ROWS_PER_TILE)])
    pltpu.sync_copy(src.at[pl.ds(src_base, nch)], src_sl)
    pltpu.sync_copy(dst.at[pl.ds(dst_base, nch)], dst_sl)
    if src_xform is not None:
        def xf(j, carry):
            for k in range(K // 16):
                idx = (j, pl.ds(16 * k, 16))
                src_sl[idx] = src_xform(src_sl[idx])
            return carry

        lax.fori_loop(0, nch, xf, 0)
    plsc.subcore_barrier()

    def g_start(j, b):
        pltpu.make_async_copy(table.at[src_sl.at[j]], rows[b], gsem[b]).start()

    def g_wait(j, b):
        pltpu.make_async_copy(table.at[src_sl.at[j]], rows[b], gsem[b]).wait()

    def s_start(j, b):
        pltpu.make_async_copy(rows[b], acc.at[dst_sl.at[j]],
                              ssem[b]).start(add=True)

    def s_wait(j, b):
        pltpu.make_async_copy(rows[b], acc.at[dst_sl.at[j]], ssem[b]).wait()

    _ring(g_start, g_wait, s_start, s_wait, nch, nbuf)
    plsc.subcore_barrier()
    return rb


def _sc_colsplit_agg():
    nch = E // NSUB // K
    nbuf = 8
    mesh = plsc.VectorSubcoreMesh(core_axis_name="c", subcore_axis_name="s")

    @functools.partial(
        pl.kernel,
        out_type=jax.ShapeDtypeStruct((N, D1), jnp.float32),
        mesh=mesh,
        scratch_types=[
            pltpu.VMEM((nch, K), jnp.int32),
            pltpu.VMEM((nch, K), jnp.int32),
            [pltpu.VMEM((K, HW1), jnp.float32)] * nbuf,
            [pltpu.SemaphoreType.DMA] * nbuf,
            [pltpu.SemaphoreType.DMA] * nbuf,
            pltpu.VMEM_SHARED((N, HW1), jnp.float32),
        ],
        compiler_params=pltpu.CompilerParams(use_tc_tiling_on_sc=False),
    )
    def agg(table, src, dst, zeros, out, src_sl, dst_sl, rows, gsem, ssem,
            acc):
        c = lax.axis_index("c")
        s = lax.axis_index("s")
        rb = _sc_ring_agg(
            table, src, dst, zeros, out, src_sl, dst_sl, rows, gsem, ssem,
            acc, src_base=s * nch, dst_base=s * nch, nch=nch, nbuf=nbuf,
            src_xform=lambda v: v * 2 + c)
        pltpu.sync_copy(acc.at[pl.ds(rb, ROWS_PER_TILE)],
                        out.at[pl.ds(rb, ROWS_PER_TILE), pl.ds(c * HW1, HW1)])

    return agg


def _sc_edgesplit_agg(width):
    nch = EDGES_PER_TILE // K
    nbuf = 8
    mesh = plsc.VectorSubcoreMesh(core_axis_name="c", subcore_axis_name="s")

    @functools.partial(
        pl.kernel,
        out_type=jax.ShapeDtypeStruct((NCORES * N, width), jnp.float32),
        mesh=mesh,
        scratch_types=[
            pltpu.VMEM((nch, K), jnp.int32),
            pltpu.VMEM((nch, K), jnp.int32),
            [pltpu.VMEM((K, width), jnp.float32)] * nbuf,
            [pltpu.SemaphoreType.DMA] * nbuf,
            [pltpu.SemaphoreType.DMA] * nbuf,
            pltpu.VMEM_SHARED((N, width), jnp.float32),
        ],
        compiler_params=pltpu.CompilerParams(use_tc_tiling_on_sc=False),
    )
    def agg(table, src, dst, zeros, out, src_sl, dst_sl, rows, gsem, ssem,
            acc):
        c = lax.axis_index("c")
        s = lax.axis_index("s")
        cb = (c * NSUB + s) * nch
        rb = _sc_ring_agg(
            table, src, dst, zeros, out, src_sl, dst_sl, rows, gsem, ssem,
            acc, src_base=cb, dst_base=cb, nch=nch, nbuf=nbuf)
        pltpu.sync_copy(acc.at[pl.ds(rb, ROWS_PER_TILE)],
                        out.at[pl.ds(c * N + rb, ROWS_PER_TILE)])

    return agg


_sc_colsplit_agg = functools.lru_cache(maxsize=None)(_sc_colsplit_agg)
_sc_edgesplit_agg = functools.lru_cache(maxsize=None)(_sc_edgesplit_agg)


def kernel(x, edge_index, batch, W1, b1, W2, b2):
    src2 = edge_index[0].reshape(E // K, K)
    dst2 = edge_index[1].reshape(E // K, K)

    t1 = pl.pallas_call(
        _layer1_body,
        grid=(N // BLK,),
        in_specs=[
            pl.BlockSpec((BLK, D1), lambda i: (i, 0)),
            pl.BlockSpec((D1, D1), lambda i: (0, 0)),
            pl.BlockSpec((1, D1), lambda i: (0, 0)),
        ],
        out_specs=pl.BlockSpec((BLK, D1), lambda i: (i, 0)),
        out_shape=jax.ShapeDtypeStruct((N, D1), jnp.float32),
    )(x, W1.T, b1.reshape(1, -1))

    zeros = jnp.zeros((N, HW1), jnp.float32)
    a1 = _sc_colsplit_agg()(t1.reshape(2 * N, HW1), src2, dst2, zeros)

    t2 = pl.pallas_call(
        _layer2_body,
        grid=(N // BLK,),
        in_specs=[
            pl.BlockSpec((BLK, D1), lambda i: (i, 0)),
            pl.BlockSpec((D1, D2), lambda i: (0, 0)),
            pl.BlockSpec((1, D2), lambda i: (0, 0)),
        ],
        out_specs=pl.BlockSpec((BLK, D2), lambda i: (i, 0)),
        out_shape=jax.ShapeDtypeStruct((N, D2), jnp.float32),
    )(a1, W2.T, b2.reshape(1, -1))

    p2 = _sc_edgesplit_agg(D2)(t2, src2, dst2, zeros)
    p2 = p2.reshape(NCORES, N, D2)

    out = pl.pallas_call(
        _pool_body,
        in_specs=[
            pl.BlockSpec((NCORES, N, D2), lambda: (0, 0, 0)),
            pl.BlockSpec((N, D1), lambda: (0, 0)),
            pl.BlockSpec((1, N), lambda: (0, 0)),
        ],
        out_specs=pl.BlockSpec((NG, D2), lambda: (0, 0)),
        out_shape=jax.ShapeDtypeStruct((NG, D2), jnp.float32),
    )(p2, a1, batch.reshape(1, -1))

    return out
